# Initial kernel scaffold; baseline (speedup 1.0000x reference)
import jax
import jax.numpy as jnp
from jax.experimental import pallas as pl


def kernel(x, tables):
    raise NotImplementedError("write your pallas kernel here")


# speedup vs baseline: 1.1440x; 1.1440x over previous
import functools

import jax
import jax.numpy as jnp
from jax import lax
from jax.experimental import pallas as pl
from jax.experimental.pallas import tpu as pltpu
from jax.experimental.pallas import tpu_sc as plsc

_NUM_FIELDS = 26
_VOCAB = 100000
_EMBED_DIM = 32
_BATCH = 16384
_ROWS = _BATCH * _NUM_FIELDS
_NW = 32
_RPW = _ROWS // _NW
_K = 1664
_NCHUNK = _RPW // _K
_NSLICE = _K // 16

_mesh = plsc.VectorSubcoreMesh(core_axis_name="c", subcore_axis_name="s")


@functools.partial(
    pl.kernel,
    mesh=_mesh,
    out_type=jax.ShapeDtypeStruct((_ROWS, _EMBED_DIM), jnp.float32),
    scratch_types=[
        pltpu.VMEM((_K,), jnp.int32),
        pltpu.VMEM((_K,), jnp.int32),
        pltpu.VMEM((_K, _EMBED_DIM), jnp.float32),
        pltpu.SemaphoreType.DMA,
    ],
    compiler_params=pltpu.CompilerParams(use_tc_tiling_on_sc=False),
)
def _emb_lookup(x_hbm, tables_hbm, off_hbm, out_hbm, idx_v, off_v, rows_v, sem):
    wid = lax.axis_index("s") * 2 + lax.axis_index("c")
    base = wid * _RPW
    pltpu.sync_copy(off_hbm, off_v)

    for c in range(_NCHUNK):
        start = base + c * _K
        pltpu.sync_copy(x_hbm.at[pl.ds(start, _K)], idx_v)

        def _add_off(i, carry):
            sl = pl.ds(i * 16, 16)
            idx_v[sl] = idx_v[sl] + off_v[sl]
            return carry

        lax.fori_loop(0, _NSLICE, _add_off, 0)

        pltpu.async_copy(tables_hbm.at[idx_v], rows_v, sem).wait()
        pltpu.sync_copy(rows_v, out_hbm.at[pl.ds(start, _K)])


def kernel(x, tables):
    x_flat = x.astype(jnp.int32).reshape(_ROWS)
    tables_flat = tables.reshape(_NUM_FIELDS * _VOCAB, _EMBED_DIM)
    off = (jnp.arange(_K, dtype=jnp.int32) % _NUM_FIELDS) * _VOCAB
    out = _emb_lookup(x_flat, tables_flat, off)
    return out.reshape(_BATCH, _NUM_FIELDS, _EMBED_DIM)


# speedup vs baseline: 1.7872x; 1.5622x over previous
import functools

import jax
import jax.numpy as jnp
from jax import lax
from jax.experimental import pallas as pl
from jax.experimental.pallas import tpu as pltpu
from jax.experimental.pallas import tpu_sc as plsc

_NUM_FIELDS = 26
_VOCAB = 100000
_EMBED_DIM = 32
_BATCH = 16384
_NPLANE = _NUM_FIELDS * _EMBED_DIM
_NW = 32
_PPW = _NPLANE // _NW
_BCH = 4096
_NCH = _BATCH // _BCH
_NSL = _BCH // 16

_mesh = plsc.VectorSubcoreMesh(core_axis_name="c", subcore_axis_name="s")


@functools.partial(
    pl.kernel,
    mesh=_mesh,
    out_type=jax.ShapeDtypeStruct((_NPLANE, _BATCH), jnp.float32),
    scratch_types=[
        pltpu.VMEM((_VOCAB,), jnp.float32),
        pltpu.VMEM((_BCH,), jnp.int32),
        pltpu.VMEM((_BCH,), jnp.float32),
    ],
    compiler_params=pltpu.CompilerParams(
        use_tc_tiling_on_sc=False, needs_layout_passes=False
    ),
)
def _emb_lookup(xt_hbm, t2_hbm, out_hbm, row_v, idx_v, val_v):
    wid = lax.axis_index("s") * 2 + lax.axis_index("c")

    def plane_body(j, carry):
        p = j * _EMBED_DIM + wid
        f = j
        pltpu.sync_copy(t2_hbm.at[p], row_v)

        def chunk_body(c, carry2):
            b0 = c * _BCH
            pltpu.sync_copy(xt_hbm.at[f, pl.ds(b0, _BCH)], idx_v)

            def gather_body(i, carry3):
                sl = pl.ds(i * 16, 16)
                val_v[sl] = plsc.load_gather(row_v, [idx_v[sl]])
                return carry3

            lax.fori_loop(0, _NSL, gather_body, 0)
            pltpu.sync_copy(val_v, out_hbm.at[p, pl.ds(b0, _BCH)])
            return carry2

        lax.fori_loop(0, _NCH, chunk_body, 0)
        return carry

    lax.fori_loop(0, _PPW, plane_body, 0)


def kernel(x, tables):
    xt = x.astype(jnp.int32).T
    t2 = tables.transpose(0, 2, 1).reshape(_NPLANE, _VOCAB)
    out = _emb_lookup(xt, t2)
    return out.reshape(_NUM_FIELDS, _EMBED_DIM, _BATCH).transpose(2, 0, 1)


# speedup vs baseline: 3.9991x; 2.2377x over previous
import functools

import jax
import jax.numpy as jnp
from jax import lax
from jax.experimental import pallas as pl
from jax.experimental.pallas import tpu as pltpu
from jax.experimental.pallas import tpu_sc as plsc

_NUM_FIELDS = 26
_VOCAB = 100000
_EMBED_DIM = 32
_BATCH = 16384
_NPLANE = _NUM_FIELDS * _EMBED_DIM
_NW = 32
_PPW = _NPLANE // _NW
_BCH = 4096
_NCH = _BATCH // _BCH
_NSL = _BCH // 16

_mesh = plsc.VectorSubcoreMesh(core_axis_name="c", subcore_axis_name="s")


@functools.partial(
    pl.kernel,
    mesh=_mesh,
    out_type=jax.ShapeDtypeStruct((_NPLANE, _BATCH), jnp.float32),
    scratch_types=[
        pltpu.VMEM((_VOCAB,), jnp.float32),
        pltpu.VMEM((_BCH,), jnp.int32),
        pltpu.VMEM((_BCH,), jnp.float32),
    ],
    compiler_params=pltpu.CompilerParams(
        use_tc_tiling_on_sc=True, needs_layout_passes=False
    ),
)
def _emb_lookup(xt_hbm, t2_hbm, out_hbm, row_v, idx_v, val_v):
    wid = lax.axis_index("s") * 2 + lax.axis_index("c")

    def plane_body(j, carry):
        p = j * _EMBED_DIM + wid
        f = j
        pltpu.sync_copy(t2_hbm.at[p], row_v)

        def chunk_body(c, carry2):
            b0 = c * _BCH
            pltpu.sync_copy(xt_hbm.at[f, pl.ds(b0, _BCH)], idx_v)

            def gather_body(i, carry3):
                sl = pl.ds(i * 16, 16)
                val_v[sl] = plsc.load_gather(row_v, [idx_v[sl]])
                return carry3

            lax.fori_loop(0, _NSL, gather_body, 0)
            pltpu.sync_copy(val_v, out_hbm.at[p, pl.ds(b0, _BCH)])
            return carry2

        lax.fori_loop(0, _NCH, chunk_body, 0)
        return carry

    lax.fori_loop(0, _PPW, plane_body, 0)


def kernel(x, tables):
    xt = x.astype(jnp.int32).T
    t2 = tables.transpose(0, 2, 1).reshape(_NPLANE, _VOCAB)
    out = _emb_lookup(xt, t2)
    return out.reshape(_NUM_FIELDS, _EMBED_DIM, _BATCH).transpose(2, 0, 1)


# speedup vs baseline: 6.0683x; 1.5174x over previous
import functools

import jax
import jax.numpy as jnp
from jax import lax
from jax.experimental import pallas as pl
from jax.experimental.pallas import tpu as pltpu
from jax.experimental.pallas import tpu_sc as plsc

_NUM_FIELDS = 26
_VOCAB = 100000
_EMBED_DIM = 32
_BATCH = 16384
_NPLANE = _NUM_FIELDS * _EMBED_DIM
_BCH = 4096
_NCH = _BATCH // _BCH
_UNROLL = 4
_NIT = _BCH // (16 * _UNROLL)

_mesh = plsc.VectorSubcoreMesh(core_axis_name="c", subcore_axis_name="s")


@functools.partial(
    pl.kernel,
    mesh=_mesh,
    out_type=jax.ShapeDtypeStruct((_NPLANE, _BATCH), jnp.float32),
    scratch_types=[
        pltpu.VMEM((_VOCAB,), jnp.float32),
        pltpu.VMEM((_BCH,), jnp.int32),
        pltpu.VMEM((_BCH,), jnp.int32),
        pltpu.VMEM((_BCH,), jnp.float32),
        pltpu.VMEM((_BCH,), jnp.float32),
        pltpu.SemaphoreType.DMA,
        pltpu.SemaphoreType.DMA,
        pltpu.SemaphoreType.DMA,
        pltpu.SemaphoreType.DMA,
        pltpu.SemaphoreType.DMA,
    ],
    compiler_params=pltpu.CompilerParams(
        use_tc_tiling_on_sc=True, needs_layout_passes=False
    ),
)
def _emb_lookup(xt_hbm, t2_hbm, out_hbm,
                row_v, idx_a, idx_b, val_a, val_b,
                s_row, s_xa, s_xb, s_oa, s_ob):
    wid = lax.axis_index("s") * 2 + lax.axis_index("c")
    idx_bufs = ((idx_a, s_xa), (idx_b, s_xb))
    val_bufs = ((val_a, s_oa), (val_b, s_ob))

    def gather_chunk(ib, vb):
        def body(i, carry):
            base = i * (16 * _UNROLL)
            for u in range(_UNROLL):
                sl = pl.ds(base + u * 16, 16)
                vb[sl] = plsc.load_gather(row_v, [ib[sl]])
            return carry

        lax.fori_loop(0, _NIT, body, 0)

    h_row = pltpu.async_copy(t2_hbm.at[wid], row_v, s_row)
    h_x = pltpu.async_copy(xt_hbm.at[0, pl.ds(0, _BCH)], idx_a, s_xa)
    out_h = [None, None]

    for j in range(_NUM_FIELDS):
        p = j * _EMBED_DIM + wid
        h_row.wait()
        for c in range(_NCH):
            ib, _ = idx_bufs[c % 2]
            vb, s_v = val_bufs[c % 2]
            h_x.wait()
            if c + 1 < _NCH:
                nib, ns = idx_bufs[(c + 1) % 2]
                h_x = pltpu.async_copy(
                    xt_hbm.at[j, pl.ds((c + 1) * _BCH, _BCH)], nib, ns)
            elif j + 1 < _NUM_FIELDS:
                nib, ns = idx_bufs[0]
                h_x = pltpu.async_copy(
                    xt_hbm.at[j + 1, pl.ds(0, _BCH)], nib, ns)
            if out_h[c % 2] is not None:
                out_h[c % 2].wait()
            gather_chunk(ib, vb)
            out_h[c % 2] = pltpu.async_copy(
                vb, out_hbm.at[p, pl.ds(c * _BCH, _BCH)], s_v)
        if j + 1 < _NUM_FIELDS:
            h_row = pltpu.async_copy(
                t2_hbm.at[(j + 1) * _EMBED_DIM + wid], row_v, s_row)

    out_h[0].wait()
    out_h[1].wait()


def kernel(x, tables):
    xt = x.astype(jnp.int32).T
    t2 = tables.transpose(0, 2, 1).reshape(_NPLANE, _VOCAB)
    out = _emb_lookup(xt, t2)
    return out.reshape(_NUM_FIELDS, _EMBED_DIM, _BATCH).transpose(2, 0, 1)
